# Initial kernel scaffold; baseline (speedup 1.0000x reference)
import jax
import jax.numpy as jnp
from jax.experimental import pallas as pl


def kernel(x, weight):
    raise NotImplementedError("write your pallas kernel here")


# speedup vs baseline: 1.7976x; 1.7976x over previous
import functools

import jax
import jax.numpy as jnp
from jax import lax
from jax.experimental import pallas as pl
from jax.experimental.pallas import tpu as pltpu
from jax.experimental.pallas import tpu_sc as plsc

_NC = 2
_NS = 16
_NW = _NC * _NS

_G = 128
_GPC = 4
_C = _G * _GPC


def _emb_body(n_chunks, table_hbm, x_hbm, out_hbm, idx_v, rows_v, gsem):
    wid = lax.axis_index("s") * _NC + lax.axis_index("c")
    per_w = n_chunks * _C
    idx_row0 = wid * (per_w // _G)
    out_row0 = wid * per_w

    def chunk(t, carry):
        pltpu.sync_copy(x_hbm.at[pl.ds(idx_row0 + t * _GPC, _GPC)], idx_v)
        cps = [
            pltpu.async_copy(
                table_hbm.at[idx_v.at[j]],
                rows_v.at[pl.ds(j * _G, _G)],
                gsem,
            )
            for j in range(_GPC)
        ]
        for cp in cps:
            cp.wait()
        pltpu.sync_copy(rows_v, out_hbm.at[pl.ds(out_row0 + t * _C, _C)])
        return carry

    lax.fori_loop(0, n_chunks, chunk, 0)


def kernel(x, weight):
    S0, S1 = x.shape
    B = S0 * S1
    D = weight.shape[1]
    assert B % (_NW * _C) == 0
    n_chunks = B // (_NW * _C)

    x2d = x.reshape(B // _G, _G).astype(jnp.int32)

    mesh = plsc.VectorSubcoreMesh(core_axis_name="c", subcore_axis_name="s")
    emb = functools.partial(
        pl.kernel,
        out_type=jax.ShapeDtypeStruct((B, D), jnp.float32),
        mesh=mesh,
        scratch_types=[
            pltpu.VMEM((_GPC, _G), jnp.int32),
            pltpu.VMEM((_C, D), jnp.float32),
            pltpu.SemaphoreType.DMA,
        ],
        compiler_params=pltpu.CompilerParams(use_tc_tiling_on_sc=False),
    )(functools.partial(_emb_body, n_chunks))

    out = emb(weight, x2d)
    return out.reshape(S0, S1, D)


# speedup vs baseline: 1.8721x; 1.0414x over previous
import functools

import jax
import jax.numpy as jnp
from jax import lax
from jax.experimental import pallas as pl
from jax.experimental.pallas import tpu as pltpu
from jax.experimental.pallas import tpu_sc as plsc

_NC = 2
_NS = 16
_NW = _NC * _NS

_G = 128
_GPC = 4
_C = _G * _GPC
_NBUF = 2


def _emb_body(n_chunks, table_hbm, x_hbm, out_hbm,
              idx_all, rows0, rows1, isem, gsem0, gsem1, osem0, osem1):
    wid = lax.axis_index("s") * _NC + lax.axis_index("c")
    per_w = n_chunks * _C
    n_idx_rows = per_w // _G
    out_row0 = wid * per_w

    pltpu.async_copy(x_hbm.at[pl.ds(wid * n_idx_rows, n_idx_rows)],
                     idx_all, isem).wait()

    bufs = ((rows0, gsem0, osem0), (rows1, gsem1, osem1))

    def pair(k, carry):
        for b in range(_NBUF):
            t = _NBUF * k + b
            rows, gsem, osem = bufs[b]

            @pl.when(k > 0)
            def _wait_prev_write():
                pltpu.make_async_copy(rows, out_hbm.at[pl.ds(0, _C)],
                                      osem).wait()

            cps = [
                pltpu.async_copy(
                    table_hbm.at[idx_all.at[t * _GPC + j]],
                    rows.at[pl.ds(j * _G, _G)],
                    gsem,
                )
                for j in range(_GPC)
            ]
            for cp in cps:
                cp.wait()
            pltpu.async_copy(rows, out_hbm.at[pl.ds(out_row0 + t * _C, _C)],
                             osem)
        return carry

    lax.fori_loop(0, n_chunks // _NBUF, pair, 0)
    for rows, _, osem in bufs:
        pltpu.make_async_copy(rows, out_hbm.at[pl.ds(0, _C)], osem).wait()


def kernel(x, weight):
    S0, S1 = x.shape
    B = S0 * S1
    D = weight.shape[1]
    assert B % (_NW * _C * _NBUF) == 0
    n_chunks = B // (_NW * _C)
    per_w = n_chunks * _C

    x2d = x.reshape(B // _G, _G).astype(jnp.int32)

    mesh = plsc.VectorSubcoreMesh(core_axis_name="c", subcore_axis_name="s")
    emb = functools.partial(
        pl.kernel,
        out_type=jax.ShapeDtypeStruct((B, D), jnp.float32),
        mesh=mesh,
        scratch_types=[
            pltpu.VMEM((per_w // _G, _G), jnp.int32),
            pltpu.VMEM((_C, D), jnp.float32),
            pltpu.VMEM((_C, D), jnp.float32),
            pltpu.SemaphoreType.DMA,
            pltpu.SemaphoreType.DMA,
            pltpu.SemaphoreType.DMA,
            pltpu.SemaphoreType.DMA,
            pltpu.SemaphoreType.DMA,
        ],
        compiler_params=pltpu.CompilerParams(use_tc_tiling_on_sc=False),
    )(functools.partial(_emb_body, n_chunks))

    out = emb(weight, x2d)
    return out.reshape(S0, S1, D)
